# Initial kernel scaffold; baseline (speedup 1.0000x reference)
import jax
import jax.numpy as jnp
from jax.experimental import pallas as pl


def kernel(x, string_embedding, params, edge_index, ppi_pairs, idx):
    raise NotImplementedError("write your pallas kernel here")


# speedup vs baseline: 1.0454x; 1.0454x over previous
import functools
import math

import jax
import jax.numpy as jnp
from jax.experimental import pallas as pl
from jax.experimental.pallas import tpu as pltpu

_F32 = jnp.float32
_BF16 = jnp.bfloat16


def _adj_body(e_ref, a_ref):
    c = pl.program_id(0)
    n = a_ref.shape[0]
    src = e_ref[0, :]
    dst = e_ref[1, :]
    cw = src.shape[0]
    oh_d = (jax.lax.broadcasted_iota(jnp.int32, (n, cw), 0) == dst[None, :]
            ).astype(_F32)
    oh_s = (jax.lax.broadcasted_iota(jnp.int32, (cw, n), 1) == src[:, None]
            ).astype(_F32)

    @pl.when(c == 0)
    def _():
        a_ref[...] = jnp.zeros_like(a_ref)

    a_ref[...] += jnp.dot(oh_d, oh_s, preferred_element_type=_F32)


def _build_adj(edge_index, n_nodes):
    e = edge_index.shape[1]
    cw = min(2048, e)
    epad = ((e + cw - 1) // cw) * cw
    ei = edge_index.astype(jnp.int32)
    if epad != e:
        ei = jnp.concatenate(
            [ei, jnp.full((2, epad - e), -1, jnp.int32)], axis=1)
    return pl.pallas_call(
        _adj_body,
        grid=(epad // cw,),
        in_specs=[pl.BlockSpec((2, cw), lambda c: (0, c))],
        out_specs=pl.BlockSpec((n_nodes, n_nodes), lambda c: (0, 0)),
        out_shape=jax.ShapeDtypeStruct((n_nodes, n_nodes), _F32),
    )(ei)


def _gin_body(x_ref, a_ref, win_ref, bin_ref,
              w1a_ref, b1a_ref, w2a_ref, b2a_ref, bga_ref, bba_ref,
              lg0_ref, lb0_ref,
              w1b_ref, b1b_ref, w2b_ref, b2b_ref, bgb_ref, bbb_ref,
              lg1_ref, lb1_ref,
              wlin_ref, blin_ref, wst_ref, bst_ref, wg1_ref, bg1_ref,
              wg2_ref, bg2_ref, eps_ref, sp_ref, g_ref):
    def mm(a, b):
        return jnp.dot(a, b, preferred_element_type=_F32)

    def ln_rows(t, g, b):
        mu = jnp.mean(t, axis=-1, keepdims=True)
        v = jnp.mean((t - mu) ** 2, axis=-1, keepdims=True)
        return (t - mu) / jnp.sqrt(v + 1e-5) * g + b

    x = x_ref[...]
    adj = a_ref[...]
    x_init = mm(x, win_ref[...]) + bin_ref[...]
    h = x
    layers = (
        (w1a_ref, b1a_ref, w2a_ref, b2a_ref, bga_ref, bba_ref, lg0_ref,
         lb0_ref),
        (w1b_ref, b1b_ref, w2b_ref, b2b_ref, bgb_ref, bbb_ref, lg1_ref,
         lb1_ref),
    )
    for l, (w1, b1, w2, b2, bg, bb, lg, lb) in enumerate(layers):
        agg = mm(adj, h)
        m = (1.0 + eps_ref[0, l]) * h + agg
        m = jnp.maximum(mm(m, w1[...]) + b1[...], 0.0)
        m = jnp.maximum(mm(m, w2[...]) + b2[...], 0.0)
        mu = jnp.mean(m, axis=0, keepdims=True)
        v = jnp.mean((m - mu) ** 2, axis=0, keepdims=True)
        m = (m - mu) / jnp.sqrt(v + 1e-5) * bg[...] + bb[...]
        h_new = m + 0.08 * x_init
        if l == 1:
            h_new = h_new + 0.1 * h
        h = ln_rows(h_new, lg[...], lb[...])
    h = jnp.maximum(mm(h, wlin_ref[...]) + blin_ref[...], 0.0)
    sp = jnp.maximum(mm(h, wst_ref[...]) + bst_ref[...], 0.0)
    g1 = jnp.maximum(mm(sp, wg1_ref[...]) + bg1_ref[...], 0.0)
    g_ref[...] = jax.nn.sigmoid(mm(g1, wg2_ref[...]) + bg2_ref[...])
    sp_ref[...] = sp


def _fusion_body(nids_ref, flags_ref, s_ref, sp_ref, g_ref, wsq_ref, bsq_ref,
                 wf_ref, bf_ref, lg_ref, lb_ref, out_ref):
    del nids_ref

    @pl.when(flags_ref[pl.program_id(1)] == 1)
    def _():
        s = s_ref[0].astype(_BF16)
        seqp = jnp.dot(s, wsq_ref[...],
                       preferred_element_type=_F32) + bsq_ref[...]
        seqp = jnp.maximum(seqp, 0.0)
        g = g_ref[0]
        spv = sp_ref[0]
        fused = g * spv + (1.0 - g) * seqp
        hrow = jnp.dot(fused.astype(_BF16), wf_ref[...],
                       preferred_element_type=_F32) + bf_ref[...]
        hrow = jnp.maximum(hrow, 0.0)
        mu = jnp.mean(hrow, axis=-1, keepdims=True)
        v = jnp.mean((hrow - mu) ** 2, axis=-1, keepdims=True)
        ln = (hrow - mu) / jnp.sqrt(v + 1e-5) * lg_ref[...] + lb_ref[...]
        nrm = jnp.sqrt(jnp.sum(ln * ln, axis=-1, keepdims=True))
        out_ref[0] = (ln / jnp.maximum(nrm, 1e-12)).astype(_BF16)


def _inter_body(pos_ref, a1_ref, a2_ref, o_ref, *, scale):
    del pos_ref
    r = jax.lax.dot_general(a1_ref[0], a2_ref[0], (((1,), (1,)), ((), ())),
                            preferred_element_type=_F32)
    o_ref[0] = r * scale


def _head_body(if_ref, w1_ref, b1_ref, w2_ref, b2_ref, w3_ref, b3_ref,
               o_ref, acc_ref, *, kt, bk):
    k = pl.program_id(0)
    nk = pl.num_programs(0)

    @pl.when(k == 0)
    def _():
        acc_ref[...] = jnp.zeros_like(acc_ref)

    def contrib(masked):
        xb = if_ref[...]
        wb = w1_ref[...]
        if masked:
            base = k * bk
            kio = jax.lax.broadcasted_iota(jnp.int32, xb.shape, 1) + base
            xb = jnp.where(kio < kt, xb, 0.0)
            wio = jax.lax.broadcasted_iota(jnp.int32, wb.shape, 0) + base
            wb = jnp.where(wio < kt, wb, 0.0)
        return jnp.dot(xb.astype(_BF16), wb.astype(_BF16),
                       preferred_element_type=_F32)

    @pl.when(k < nk - 1)
    def _():
        acc_ref[...] += contrib(False)

    @pl.when(k == nk - 1)
    def _():
        acc = acc_ref[...] + contrib(True)
        z1 = jnp.maximum(acc + b1_ref[...], 0.0)
        z2 = jnp.maximum(
            jnp.dot(z1, w2_ref[...], preferred_element_type=_F32)
            + b2_ref[...], 0.0)
        o_ref[...] = (jnp.sum(z2 * w3_ref[...], axis=-1, keepdims=True)
                      + b3_ref[...])


def kernel(x, string_embedding, params, edge_index, ppi_pairs, idx):
    n, hid = x.shape
    _, seq, d_esm = string_embedding.shape
    b = idx.shape[0]
    p = params
    fus = p['fusion']
    head = p['head']
    ff = fus['Wst'].shape[1]

    def r2(v):
        return v.reshape(1, -1)

    pairs = jnp.take(ppi_pairs, idx, axis=0)
    nids = jnp.concatenate([pairs[:, 0], pairs[:, 1]]).astype(jnp.int32)
    snids = jnp.sort(nids)
    first = jnp.concatenate([
        jnp.ones((1,), jnp.int32),
        (snids[1:] != snids[:-1]).astype(jnp.int32)])
    pos = jnp.searchsorted(snids, nids).astype(jnp.int32)
    posm = jnp.stack([pos[:b], pos[b:]])

    adj = _build_adj(edge_index, n)

    gin = p['gin']
    eps2 = jnp.stack([gin[0]['eps'], gin[1]['eps']]).reshape(1, 2)
    sp, gt = pl.pallas_call(
        _gin_body,
        out_shape=(jax.ShapeDtypeStruct((n, ff), _F32),
                   jax.ShapeDtypeStruct((n, ff), _F32)),
    )(x, adj, p['W_in'], r2(p['b_in']),
      gin[0]['W1'], r2(gin[0]['b1']), gin[0]['W2'], r2(gin[0]['b2']),
      r2(gin[0]['bn_g']), r2(gin[0]['bn_b']),
      r2(p['ln_g'][0]), r2(p['ln_b'][0]),
      gin[1]['W1'], r2(gin[1]['b1']), gin[1]['W2'], r2(gin[1]['b2']),
      r2(gin[1]['bn_g']), r2(gin[1]['bn_b']),
      r2(p['ln_g'][1]), r2(p['ln_b'][1]),
      p['W_lin'], r2(p['b_lin']), fus['Wst'], r2(fus['bst']),
      fus['Wg1'], r2(fus['bg1']), fus['Wg2'], r2(fus['bg2']), eps2)

    nn = 2 * b
    ts = min(128, seq)
    jt = (seq + ts - 1) // ts
    sp3 = sp.reshape(n, 1, ff)
    g3 = gt.reshape(n, 1, ff)
    cidx = lambda j, i, nids_s, flags_s: (0, 0)
    fusion_spec = pltpu.PrefetchScalarGridSpec(
        num_scalar_prefetch=2,
        grid=(jt, nn),
        in_specs=[
            pl.BlockSpec((1, ts, d_esm),
                         lambda j, i, nids_s, flags_s: (nids_s[i], j, 0)),
            pl.BlockSpec((1, 1, ff),
                         lambda j, i, nids_s, flags_s: (nids_s[i], 0, 0)),
            pl.BlockSpec((1, 1, ff),
                         lambda j, i, nids_s, flags_s: (nids_s[i], 0, 0)),
            pl.BlockSpec((d_esm, ff), cidx),
            pl.BlockSpec((1, ff), cidx),
            pl.BlockSpec((ff, ff), cidx),
            pl.BlockSpec((1, ff), cidx),
            pl.BlockSpec((1, ff), cidx),
            pl.BlockSpec((1, ff), cidx),
        ],
        out_specs=pl.BlockSpec((1, ts, ff),
                               lambda j, i, nids_s, flags_s: (i, j, 0)),
    )
    a = pl.pallas_call(
        _fusion_body,
        grid_spec=fusion_spec,
        out_shape=jax.ShapeDtypeStruct((nn, seq, ff), _BF16),
    )(snids, first, string_embedding, sp3, g3,
      fus['Wsq'].astype(_BF16), r2(fus['bsq']), fus['Wf'].astype(_BF16),
      r2(fus['bf']), r2(fus['ln_g']), r2(fus['ln_b']))

    inter_spec = pltpu.PrefetchScalarGridSpec(
        num_scalar_prefetch=1,
        grid=(b,),
        in_specs=[
            pl.BlockSpec((1, seq, ff), lambda i, pos_s: (pos_s[0, i], 0, 0)),
            pl.BlockSpec((1, seq, ff), lambda i, pos_s: (pos_s[1, i], 0, 0)),
        ],
        out_specs=pl.BlockSpec((1, seq, seq), lambda i, pos_s: (i, 0, 0)),
    )
    inter = pl.pallas_call(
        functools.partial(_inter_body, scale=1.0 / math.sqrt(ff)),
        grid_spec=inter_spec,
        out_shape=jax.ShapeDtypeStruct((b, seq, seq), _F32),
    )(posm, a, a)

    kt = seq * seq
    bk = min(4096, kt)
    nkb = (kt + bk - 1) // bk
    fh = head['W1'].shape[1]
    f2 = head['W2'].shape[1]
    chead = lambda k: (0, 0)
    out = pl.pallas_call(
        functools.partial(_head_body, kt=kt, bk=bk),
        grid=(nkb,),
        in_specs=[
            pl.BlockSpec((b, bk), lambda k: (0, k)),
            pl.BlockSpec((bk, fh), lambda k: (k, 0)),
            pl.BlockSpec((1, fh), chead),
            pl.BlockSpec((fh, f2), chead),
            pl.BlockSpec((1, f2), chead),
            pl.BlockSpec((1, f2), chead),
            pl.BlockSpec((1, 1), chead),
        ],
        out_specs=pl.BlockSpec((b, 1), chead),
        out_shape=jax.ShapeDtypeStruct((b, 1), _F32),
        scratch_shapes=[pltpu.VMEM((b, fh), _F32)],
    )(inter.reshape(b, kt), head['W1'], r2(head['b1']), head['W2'],
      r2(head['b2']), head['W3'].reshape(1, -1), head['b3'].reshape(1, 1))

    return out, inter


# speedup vs baseline: 1.1472x; 1.0974x over previous
import functools
import math

import jax
import jax.numpy as jnp
from jax.experimental import pallas as pl
from jax.experimental.pallas import tpu as pltpu

_F32 = jnp.float32
_BF16 = jnp.bfloat16


def _adj_body(e_ref, a_ref):
    c = pl.program_id(0)
    n = a_ref.shape[0]
    src = e_ref[0, :]
    dst = e_ref[1, :]
    cw = src.shape[0]
    oh_d = (jax.lax.broadcasted_iota(jnp.int32, (n, cw), 0) == dst[None, :]
            ).astype(_F32)
    oh_s = (jax.lax.broadcasted_iota(jnp.int32, (cw, n), 1) == src[:, None]
            ).astype(_F32)

    @pl.when(c == 0)
    def _():
        a_ref[...] = jnp.zeros_like(a_ref)

    a_ref[...] += jnp.dot(oh_d, oh_s, preferred_element_type=_F32)


def _build_adj(edge_index, n_nodes):
    e = edge_index.shape[1]
    cw = min(2048, e)
    epad = ((e + cw - 1) // cw) * cw
    ei = edge_index.astype(jnp.int32)
    if epad != e:
        ei = jnp.concatenate(
            [ei, jnp.full((2, epad - e), -1, jnp.int32)], axis=1)
    return pl.pallas_call(
        _adj_body,
        grid=(epad // cw,),
        in_specs=[pl.BlockSpec((2, cw), lambda c: (0, c))],
        out_specs=pl.BlockSpec((n_nodes, n_nodes), lambda c: (0, 0)),
        out_shape=jax.ShapeDtypeStruct((n_nodes, n_nodes), _F32),
    )(ei)


def _gin_body(x_ref, a_ref, win_ref, bin_ref,
              w1a_ref, b1a_ref, w2a_ref, b2a_ref, bga_ref, bba_ref,
              lg0_ref, lb0_ref,
              w1b_ref, b1b_ref, w2b_ref, b2b_ref, bgb_ref, bbb_ref,
              lg1_ref, lb1_ref,
              wlin_ref, blin_ref, wst_ref, bst_ref, wg1_ref, bg1_ref,
              wg2_ref, bg2_ref, eps_ref, sp_ref, g_ref):
    def mm(a, b):
        return jnp.dot(a, b, preferred_element_type=_F32)

    def seg_mm(a, h):
        hh = h.astype(_BF16)
        hl = (h - hh.astype(_F32)).astype(_BF16)
        return mm(a, hh) + mm(a, hl)

    def ln_rows(t, g, b):
        mu = jnp.mean(t, axis=-1, keepdims=True)
        v = jnp.mean((t - mu) ** 2, axis=-1, keepdims=True)
        return (t - mu) / jnp.sqrt(v + 1e-5) * g + b

    x = x_ref[...]
    adj = a_ref[...]
    x_init = mm(x, win_ref[...]) + bin_ref[...]
    h = x
    layers = (
        (w1a_ref, b1a_ref, w2a_ref, b2a_ref, bga_ref, bba_ref, lg0_ref,
         lb0_ref),
        (w1b_ref, b1b_ref, w2b_ref, b2b_ref, bgb_ref, bbb_ref, lg1_ref,
         lb1_ref),
    )
    for l, (w1, b1, w2, b2, bg, bb, lg, lb) in enumerate(layers):
        agg = seg_mm(adj, h)
        m = (1.0 + eps_ref[0, l]) * h + agg
        m = jnp.maximum(mm(m, w1[...]) + b1[...], 0.0)
        m = jnp.maximum(mm(m, w2[...]) + b2[...], 0.0)
        mu = jnp.mean(m, axis=0, keepdims=True)
        v = jnp.mean((m - mu) ** 2, axis=0, keepdims=True)
        m = (m - mu) / jnp.sqrt(v + 1e-5) * bg[...] + bb[...]
        h_new = m + 0.08 * x_init
        if l == 1:
            h_new = h_new + 0.1 * h
        h = ln_rows(h_new, lg[...], lb[...])
    h = jnp.maximum(mm(h, wlin_ref[...]) + blin_ref[...], 0.0)
    sp = jnp.maximum(mm(h, wst_ref[...]) + bst_ref[...], 0.0)
    g1 = jnp.maximum(mm(sp, wg1_ref[...]) + bg1_ref[...], 0.0)
    g_ref[...] = jax.nn.sigmoid(mm(g1, wg2_ref[...]) + bg2_ref[...])[:, None, :]
    sp_ref[...] = sp[:, None, :]


def _fusion_body(nids_ref, flags_ref, s_ref, sp_ref, g_ref, wsq_ref, bsq_ref,
                 wf_ref, bf_ref, lg_ref, lb_ref, out_ref):
    del nids_ref

    @pl.when(flags_ref[pl.program_id(0)] == 1)
    def _():
        s = s_ref[0].astype(_BF16)
        seqp = jnp.dot(s, wsq_ref[...],
                       preferred_element_type=_F32) + bsq_ref[...]
        seqp = jnp.maximum(seqp, 0.0)
        g = g_ref[0]
        spv = sp_ref[0]
        fused = g * spv + (1.0 - g) * seqp
        hrow = jnp.dot(fused.astype(_BF16), wf_ref[...],
                       preferred_element_type=_F32) + bf_ref[...]
        hrow = jnp.maximum(hrow, 0.0)
        mu = jnp.mean(hrow, axis=-1, keepdims=True)
        v = jnp.mean((hrow - mu) ** 2, axis=-1, keepdims=True)
        ln = (hrow - mu) / jnp.sqrt(v + 1e-5) * lg_ref[...] + lb_ref[...]
        nrm = jnp.sqrt(jnp.sum(ln * ln, axis=-1, keepdims=True))
        out_ref[0] = ln / jnp.maximum(nrm, 1e-12)


def _inter_body(pos_ref, a1_ref, a2_ref, o_ref, *, scale):
    del pos_ref
    r = jax.lax.dot_general(a1_ref[0], a2_ref[0], (((1,), (1,)), ((), ())),
                            preferred_element_type=_F32)
    o_ref[0] = r * scale


def _head_body(if_ref, w1_ref, b1_ref, w2_ref, b2_ref, w3_ref, b3_ref,
               o_ref, acc_ref, *, ls, seq_total):
    k = pl.program_id(0)
    nk = pl.num_programs(0)

    @pl.when(k == 0)
    def _():
        acc_ref[...] = jnp.zeros_like(acc_ref)

    def contrib(masked):
        total = None
        for i in range(ls):
            xrow = if_ref[:, i, :]
            wrow = w1_ref[i]
            if masked:
                valid = (k * ls + i) < seq_total
                xrow = jnp.where(valid, xrow, 0.0)
                wrow = jnp.where(valid, wrow, 0.0)
            t = jnp.dot(xrow, wrow, preferred_element_type=_F32)
            total = t if total is None else total + t
        return total

    @pl.when(k < nk - 1)
    def _():
        acc_ref[...] += contrib(False)

    @pl.when(k == nk - 1)
    def _():
        acc = acc_ref[...] + contrib(True)
        z1 = jnp.maximum(acc + b1_ref[...], 0.0)
        z2 = jnp.maximum(
            jnp.dot(z1, w2_ref[...], preferred_element_type=_F32)
            + b2_ref[...], 0.0)
        o_ref[...] = (jnp.sum(z2 * w3_ref[...], axis=-1, keepdims=True)
                      + b3_ref[...])


def kernel(x, string_embedding, params, edge_index, ppi_pairs, idx):
    n, hid = x.shape
    _, seq, d_esm = string_embedding.shape
    b = idx.shape[0]
    p = params
    fus = p['fusion']
    head = p['head']
    ff = fus['Wst'].shape[1]

    def r2(v):
        return v.reshape(1, -1)

    pairs = jnp.take(ppi_pairs, idx, axis=0)
    nids = jnp.concatenate([pairs[:, 0], pairs[:, 1]]).astype(jnp.int32)
    snids = jnp.sort(nids)
    first = jnp.concatenate([
        jnp.ones((1,), jnp.int32),
        (snids[1:] != snids[:-1]).astype(jnp.int32)])
    pos = jnp.searchsorted(snids, nids).astype(jnp.int32)
    posm = jnp.stack([pos[:b], pos[b:]])

    adj = _build_adj(edge_index, n)

    gin = p['gin']
    eps2 = jnp.stack([gin[0]['eps'], gin[1]['eps']]).reshape(1, 2)
    sp3, g3 = pl.pallas_call(
        _gin_body,
        out_shape=(jax.ShapeDtypeStruct((n, 1, ff), _F32),
                   jax.ShapeDtypeStruct((n, 1, ff), _F32)),
    )(x, adj, p['W_in'], r2(p['b_in']),
      gin[0]['W1'], r2(gin[0]['b1']), gin[0]['W2'], r2(gin[0]['b2']),
      r2(gin[0]['bn_g']), r2(gin[0]['bn_b']),
      r2(p['ln_g'][0]), r2(p['ln_b'][0]),
      gin[1]['W1'], r2(gin[1]['b1']), gin[1]['W2'], r2(gin[1]['b2']),
      r2(gin[1]['bn_g']), r2(gin[1]['bn_b']),
      r2(p['ln_g'][1]), r2(p['ln_b'][1]),
      p['W_lin'], r2(p['b_lin']), fus['Wst'], r2(fus['bst']),
      fus['Wg1'], r2(fus['bg1']), fus['Wg2'], r2(fus['bg2']), eps2)

    nn = 2 * b
    cidx = lambda i, nids_s, flags_s: (0, 0)
    fusion_spec = pltpu.PrefetchScalarGridSpec(
        num_scalar_prefetch=2,
        grid=(nn,),
        in_specs=[
            pl.BlockSpec((1, seq, d_esm),
                         lambda i, nids_s, flags_s: (nids_s[i], 0, 0)),
            pl.BlockSpec((1, 1, ff),
                         lambda i, nids_s, flags_s: (nids_s[i], 0, 0)),
            pl.BlockSpec((1, 1, ff),
                         lambda i, nids_s, flags_s: (nids_s[i], 0, 0)),
            pl.BlockSpec((d_esm, ff), cidx),
            pl.BlockSpec((1, ff), cidx),
            pl.BlockSpec((ff, ff), cidx),
            pl.BlockSpec((1, ff), cidx),
            pl.BlockSpec((1, ff), cidx),
            pl.BlockSpec((1, ff), cidx),
        ],
        out_specs=pl.BlockSpec((1, seq, ff),
                               lambda i, nids_s, flags_s: (i, 0, 0)),
    )
    a = pl.pallas_call(
        _fusion_body,
        grid_spec=fusion_spec,
        out_shape=jax.ShapeDtypeStruct((nn, seq, ff), _F32),
    )(snids, first, string_embedding, sp3, g3,
      fus['Wsq'].astype(_BF16), r2(fus['bsq']), fus['Wf'].astype(_BF16),
      r2(fus['bf']), r2(fus['ln_g']), r2(fus['ln_b']))

    inter_spec = pltpu.PrefetchScalarGridSpec(
        num_scalar_prefetch=1,
        grid=(b,),
        in_specs=[
            pl.BlockSpec((1, seq, ff), lambda i, pos_s: (pos_s[0, i], 0, 0)),
            pl.BlockSpec((1, seq, ff), lambda i, pos_s: (pos_s[1, i], 0, 0)),
        ],
        out_specs=pl.BlockSpec((1, seq, seq), lambda i, pos_s: (i, 0, 0)),
    )
    inter = pl.pallas_call(
        functools.partial(_inter_body, scale=1.0 / math.sqrt(ff)),
        grid_spec=inter_spec,
        out_shape=jax.ShapeDtypeStruct((b, seq, seq), _F32),
    )(posm, a, a)

    ls = min(8, seq)
    nkb = (seq + ls - 1) // ls
    fh = head['W1'].shape[1]
    f2 = head['W2'].shape[1]
    w1r = head['W1'].reshape(seq, seq, fh)
    chead = lambda k: (0, 0)
    out = pl.pallas_call(
        functools.partial(_head_body, ls=ls, seq_total=seq),
        grid=(nkb,),
        in_specs=[
            pl.BlockSpec((b, ls, seq), lambda k: (0, k, 0)),
            pl.BlockSpec((ls, seq, fh), lambda k: (k, 0, 0)),
            pl.BlockSpec((1, fh), chead),
            pl.BlockSpec((fh, f2), chead),
            pl.BlockSpec((1, f2), chead),
            pl.BlockSpec((1, f2), chead),
            pl.BlockSpec((1, 1), chead),
        ],
        out_specs=pl.BlockSpec((b, 1), chead),
        out_shape=jax.ShapeDtypeStruct((b, 1), _F32),
        scratch_shapes=[pltpu.VMEM((b, fh), _F32)],
    )(inter, w1r, r2(head['b1']), head['W2'],
      r2(head['b2']), head['W3'].reshape(1, -1), head['b3'].reshape(1, 1))

    return out, inter


# speedup vs baseline: 1.1491x; 1.0016x over previous
import functools
import math

import jax
import jax.numpy as jnp
from jax.experimental import pallas as pl
from jax.experimental.pallas import tpu as pltpu

_F32 = jnp.float32
_BF16 = jnp.bfloat16


def _adj_body(e_ref, a_ref):
    c = pl.program_id(0)
    n = a_ref.shape[0]
    src = e_ref[0, :]
    dst = e_ref[1, :]
    cw = src.shape[0]
    oh_d = (jax.lax.broadcasted_iota(jnp.int32, (n, cw), 0) == dst[None, :]
            ).astype(_F32)
    oh_s = (jax.lax.broadcasted_iota(jnp.int32, (cw, n), 1) == src[:, None]
            ).astype(_F32)

    @pl.when(c == 0)
    def _():
        a_ref[...] = jnp.zeros_like(a_ref)

    a_ref[...] += jnp.dot(oh_d, oh_s, preferred_element_type=_F32)


def _build_adj(edge_index, n_nodes):
    e = edge_index.shape[1]
    cw = min(2048, e)
    epad = ((e + cw - 1) // cw) * cw
    ei = edge_index.astype(jnp.int32)
    if epad != e:
        ei = jnp.concatenate(
            [ei, jnp.full((2, epad - e), -1, jnp.int32)], axis=1)
    return pl.pallas_call(
        _adj_body,
        grid=(epad // cw,),
        in_specs=[pl.BlockSpec((2, cw), lambda c: (0, c))],
        out_specs=pl.BlockSpec((n_nodes, n_nodes), lambda c: (0, 0)),
        out_shape=jax.ShapeDtypeStruct((n_nodes, n_nodes), _F32),
    )(ei)


def _gin_body(x_ref, a_ref, win_ref, bin_ref,
              w1a_ref, b1a_ref, w2a_ref, b2a_ref, bga_ref, bba_ref,
              lg0_ref, lb0_ref,
              w1b_ref, b1b_ref, w2b_ref, b2b_ref, bgb_ref, bbb_ref,
              lg1_ref, lb1_ref,
              wlin_ref, blin_ref, wst_ref, bst_ref, wg1_ref, bg1_ref,
              wg2_ref, bg2_ref, eps_ref, sp_ref, g_ref):
    def mm(a, b):
        return jnp.dot(a, b, preferred_element_type=_F32)

    def seg_mm(a, h):
        hh = h.astype(_BF16)
        hl = (h - hh.astype(_F32)).astype(_BF16)
        return mm(a, hh) + mm(a, hl)

    def ln_rows(t, g, b):
        mu = jnp.mean(t, axis=-1, keepdims=True)
        v = jnp.mean((t - mu) ** 2, axis=-1, keepdims=True)
        return (t - mu) / jnp.sqrt(v + 1e-5) * g + b

    x = x_ref[...]
    adj = a_ref[...]
    x_init = mm(x, win_ref[...]) + bin_ref[...]
    h = x
    layers = (
        (w1a_ref, b1a_ref, w2a_ref, b2a_ref, bga_ref, bba_ref, lg0_ref,
         lb0_ref),
        (w1b_ref, b1b_ref, w2b_ref, b2b_ref, bgb_ref, bbb_ref, lg1_ref,
         lb1_ref),
    )
    for l, (w1, b1, w2, b2, bg, bb, lg, lb) in enumerate(layers):
        agg = seg_mm(adj, h)
        m = (1.0 + eps_ref[0, l]) * h + agg
        m = jnp.maximum(mm(m, w1[...]) + b1[...], 0.0)
        m = jnp.maximum(mm(m, w2[...]) + b2[...], 0.0)
        mu = jnp.mean(m, axis=0, keepdims=True)
        v = jnp.mean((m - mu) ** 2, axis=0, keepdims=True)
        m = (m - mu) / jnp.sqrt(v + 1e-5) * bg[...] + bb[...]
        h_new = m + 0.08 * x_init
        if l == 1:
            h_new = h_new + 0.1 * h
        h = ln_rows(h_new, lg[...], lb[...])
    h = jnp.maximum(mm(h, wlin_ref[...]) + blin_ref[...], 0.0)
    sp = jnp.maximum(mm(h, wst_ref[...]) + bst_ref[...], 0.0)
    g1 = jnp.maximum(mm(sp, wg1_ref[...]) + bg1_ref[...], 0.0)
    g_ref[...] = jax.nn.sigmoid(mm(g1, wg2_ref[...]) + bg2_ref[...])[:, None, :]
    sp_ref[...] = sp[:, None, :]


def _fusion_body(nids_ref, flags_ref, s_ref, sp_ref, g_ref, wsq_ref, bsq_ref,
                 wf_ref, bf_ref, lg_ref, lb_ref, out_ref):
    i = pl.program_id(0)

    @pl.when(flags_ref[i] == 1)
    def _():
        nid = nids_ref[i]
        s = s_ref[0].astype(_BF16)
        seqp = jnp.dot(s, wsq_ref[...],
                       preferred_element_type=_F32) + bsq_ref[...]
        seqp = jnp.maximum(seqp, 0.0)
        g = g_ref[nid]
        spv = sp_ref[nid]
        fused = g * spv + (1.0 - g) * seqp
        hrow = jnp.dot(fused.astype(_BF16), wf_ref[...],
                       preferred_element_type=_F32) + bf_ref[...]
        hrow = jnp.maximum(hrow, 0.0)
        mu = jnp.mean(hrow, axis=-1, keepdims=True)
        v = jnp.mean((hrow - mu) ** 2, axis=-1, keepdims=True)
        ln = (hrow - mu) / jnp.sqrt(v + 1e-5) * lg_ref[...] + lb_ref[...]
        nrm = jnp.sqrt(jnp.sum(ln * ln, axis=-1, keepdims=True))
        out_ref[0] = ln / jnp.maximum(nrm, 1e-12)


def _inter_body(pos_ref, a1_ref, a2_ref, o_ref, *, scale):
    del pos_ref
    r = jax.lax.dot_general(a1_ref[0], a2_ref[0], (((1,), (1,)), ((), ())),
                            preferred_element_type=_F32)
    o_ref[0] = r * scale


def _head_body(if_ref, w1_ref, b1_ref, w2_ref, b2_ref, w3_ref, b3_ref,
               o_ref, acc_ref, *, ls, seq_total):
    k = pl.program_id(0)
    nk = pl.num_programs(0)

    @pl.when(k == 0)
    def _():
        acc_ref[...] = jnp.zeros_like(acc_ref)

    def contrib(masked):
        total = None
        for i in range(ls):
            xrow = if_ref[:, i, :]
            wrow = w1_ref[i]
            if masked:
                valid = (k * ls + i) < seq_total
                xrow = jnp.where(valid, xrow, 0.0)
                wrow = jnp.where(valid, wrow, 0.0)
            t = jnp.dot(xrow, wrow, preferred_element_type=_F32)
            total = t if total is None else total + t
        return total

    @pl.when(k < nk - 1)
    def _():
        acc_ref[...] += contrib(False)

    @pl.when(k == nk - 1)
    def _():
        acc = acc_ref[...] + contrib(True)
        z1 = jnp.maximum(acc + b1_ref[...], 0.0)
        z2 = jnp.maximum(
            jnp.dot(z1, w2_ref[...], preferred_element_type=_F32)
            + b2_ref[...], 0.0)
        o_ref[...] = (jnp.sum(z2 * w3_ref[...], axis=-1, keepdims=True)
                      + b3_ref[...])


def kernel(x, string_embedding, params, edge_index, ppi_pairs, idx):
    n, hid = x.shape
    _, seq, d_esm = string_embedding.shape
    b = idx.shape[0]
    p = params
    fus = p['fusion']
    head = p['head']
    ff = fus['Wst'].shape[1]

    def r2(v):
        return v.reshape(1, -1)

    pairs = jnp.take(ppi_pairs, idx, axis=0)
    nids = jnp.concatenate([pairs[:, 0], pairs[:, 1]]).astype(jnp.int32)
    snids = jnp.sort(nids)
    first = jnp.concatenate([
        jnp.ones((1,), jnp.int32),
        (snids[1:] != snids[:-1]).astype(jnp.int32)])
    pos = jnp.searchsorted(snids, nids).astype(jnp.int32)
    posm = jnp.stack([pos[:b], pos[b:]])

    adj = _build_adj(edge_index, n)

    gin = p['gin']
    eps2 = jnp.stack([gin[0]['eps'], gin[1]['eps']]).reshape(1, 2)
    sp3, g3 = pl.pallas_call(
        _gin_body,
        out_shape=(jax.ShapeDtypeStruct((n, 1, ff), _F32),
                   jax.ShapeDtypeStruct((n, 1, ff), _F32)),
    )(x, adj, p['W_in'], r2(p['b_in']),
      gin[0]['W1'], r2(gin[0]['b1']), gin[0]['W2'], r2(gin[0]['b2']),
      r2(gin[0]['bn_g']), r2(gin[0]['bn_b']),
      r2(p['ln_g'][0]), r2(p['ln_b'][0]),
      gin[1]['W1'], r2(gin[1]['b1']), gin[1]['W2'], r2(gin[1]['b2']),
      r2(gin[1]['bn_g']), r2(gin[1]['bn_b']),
      r2(p['ln_g'][1]), r2(p['ln_b'][1]),
      p['W_lin'], r2(p['b_lin']), fus['Wst'], r2(fus['bst']),
      fus['Wg1'], r2(fus['bg1']), fus['Wg2'], r2(fus['bg2']), eps2)

    nn = 2 * b
    cidx = lambda i, nids_s, flags_s: (0, 0)
    fusion_spec = pltpu.PrefetchScalarGridSpec(
        num_scalar_prefetch=2,
        grid=(nn,),
        in_specs=[
            pl.BlockSpec((1, seq, d_esm),
                         lambda i, nids_s, flags_s: (nids_s[i], 0, 0)),
            pl.BlockSpec((n, 1, ff), lambda i, nids_s, flags_s: (0, 0, 0)),
            pl.BlockSpec((n, 1, ff), lambda i, nids_s, flags_s: (0, 0, 0)),
            pl.BlockSpec((d_esm, ff), cidx),
            pl.BlockSpec((1, ff), cidx),
            pl.BlockSpec((ff, ff), cidx),
            pl.BlockSpec((1, ff), cidx),
            pl.BlockSpec((1, ff), cidx),
            pl.BlockSpec((1, ff), cidx),
        ],
        out_specs=pl.BlockSpec((1, seq, ff),
                               lambda i, nids_s, flags_s: (i, 0, 0)),
    )
    a = pl.pallas_call(
        _fusion_body,
        grid_spec=fusion_spec,
        out_shape=jax.ShapeDtypeStruct((nn, seq, ff), _F32),
    )(snids, first, string_embedding, sp3, g3,
      fus['Wsq'].astype(_BF16), r2(fus['bsq']), fus['Wf'].astype(_BF16),
      r2(fus['bf']), r2(fus['ln_g']), r2(fus['ln_b']))

    inter_spec = pltpu.PrefetchScalarGridSpec(
        num_scalar_prefetch=1,
        grid=(b,),
        in_specs=[
            pl.BlockSpec((1, seq, ff), lambda i, pos_s: (pos_s[0, i], 0, 0)),
            pl.BlockSpec((1, seq, ff), lambda i, pos_s: (pos_s[1, i], 0, 0)),
        ],
        out_specs=pl.BlockSpec((1, seq, seq), lambda i, pos_s: (i, 0, 0)),
    )
    inter = pl.pallas_call(
        functools.partial(_inter_body, scale=1.0 / math.sqrt(ff)),
        grid_spec=inter_spec,
        out_shape=jax.ShapeDtypeStruct((b, seq, seq), _F32),
    )(posm, a, a)

    ls = min(8, seq)
    nkb = (seq + ls - 1) // ls
    fh = head['W1'].shape[1]
    f2 = head['W2'].shape[1]
    w1r = head['W1'].reshape(seq, seq, fh)
    chead = lambda k: (0, 0)
    out = pl.pallas_call(
        functools.partial(_head_body, ls=ls, seq_total=seq),
        grid=(nkb,),
        in_specs=[
            pl.BlockSpec((b, ls, seq), lambda k: (0, k, 0)),
            pl.BlockSpec((ls, seq, fh), lambda k: (k, 0, 0)),
            pl.BlockSpec((1, fh), chead),
            pl.BlockSpec((fh, f2), chead),
            pl.BlockSpec((1, f2), chead),
            pl.BlockSpec((1, f2), chead),
            pl.BlockSpec((1, 1), chead),
        ],
        out_specs=pl.BlockSpec((b, 1), chead),
        out_shape=jax.ShapeDtypeStruct((b, 1), _F32),
        scratch_shapes=[pltpu.VMEM((b, fh), _F32)],
    )(inter, w1r, r2(head['b1']), head['W2'],
      r2(head['b2']), head['W3'].reshape(1, -1), head['b3'].reshape(1, 1))

    return out, inter
